# baseline (speedup 1.0000x reference)
import functools

import jax
import jax.numpy as jnp
from jax import lax
from jax.experimental import pallas as pl
from jax.experimental.pallas import tpu as pltpu
from jax.experimental.pallas import tpu_sc as plsc

_N = 10000
_E = 320000
_D = 128
_L = 5
_G = 64
_C = 10

_NC = 2
_NS = 16
_NW = _NC * _NS
_EPW = _E // _NW
_K = 80
_NCHUNK = _EPW // _K
_NPAD = 10240
_RPT = _NPAD // _NS


def _agg_body(h_hbm, src_hbm, dst_hbm, zero_hbm,
              out0_hbm, out1_hbm,
              acc, src_v, dst_v, rows0, rows1, gsem0, gsem1, zsem):
    c = lax.axis_index("c")
    s = lax.axis_index("s")
    wid = c * _NS + s

    cp_src = pltpu.async_copy(src_hbm.at[pl.ds(wid * _EPW, _EPW)], src_v, gsem0)
    cp_dst = pltpu.async_copy(dst_hbm.at[wid], dst_v, gsem1)
    cp_zero = pltpu.async_copy(zero_hbm.at[pl.ds(s * _RPT, _RPT)],
                               acc.at[pl.ds(s * _RPT, _RPT)], zsem)
    cp_src.wait()
    pltpu.async_copy(h_hbm.at[src_v.at[pl.ds(0, _K)]], rows0, gsem0)
    cp_dst.wait()
    cp_zero.wait()
    plsc.subcore_barrier()

    def pair(i, carry):
        t = 2 * i
        pltpu.async_copy(h_hbm.at[src_v.at[pl.ds((t + 1) * _K, _K)]], rows1, gsem1)
        pltpu.make_async_copy(h_hbm.at[src_v.at[pl.ds(t * _K, _K)]], rows0, gsem0).wait()
        pltpu.sync_copy(rows0, acc.at[dst_v.at[t]], add=True)
        pltpu.async_copy(h_hbm.at[src_v.at[pl.ds((t + 2) * _K, _K)]], rows0, gsem0)
        pltpu.make_async_copy(h_hbm.at[src_v.at[pl.ds((t + 1) * _K, _K)]], rows1, gsem1).wait()
        pltpu.sync_copy(rows1, acc.at[dst_v.at[t + 1]], add=True)
        return carry

    lax.fori_loop(0, (_NCHUNK - 1) // 2, pair, 0)
    pltpu.make_async_copy(h_hbm.at[src_v.at[pl.ds((_NCHUNK - 1) * _K, _K)]], rows0, gsem0).wait()
    pltpu.sync_copy(rows0, acc.at[dst_v.at[_NCHUNK - 1]], add=True)
    plsc.subcore_barrier()

    @pl.when(c == 0)
    def _():
        pltpu.sync_copy(acc.at[pl.ds(s * _RPT, _RPT)],
                        out0_hbm.at[pl.ds(s * _RPT, _RPT)])

    @pl.when(c == 1)
    def _():
        pltpu.sync_copy(acc.at[pl.ds(s * _RPT, _RPT)],
                        out1_hbm.at[pl.ds(s * _RPT, _RPT)])


@functools.cache
def _make_agg():
    return pl.kernel(
        _agg_body,
        out_type=(jax.ShapeDtypeStruct((_NPAD, _D), jnp.float32),
                  jax.ShapeDtypeStruct((_NPAD, _D), jnp.float32)),
        mesh=plsc.VectorSubcoreMesh(core_axis_name="c", subcore_axis_name="s",
                                    num_cores=_NC, num_subcores=_NS),
        scratch_types=[
            pltpu.VMEM_SHARED((_NPAD, _D), jnp.float32),
            pltpu.VMEM((_EPW,), jnp.int32),
            pltpu.VMEM((_NCHUNK, _K), jnp.int32),
            pltpu.VMEM((_K, _D), jnp.float32),
            pltpu.VMEM((_K, _D), jnp.float32),
            pltpu.SemaphoreType.DMA,
            pltpu.SemaphoreType.DMA,
            pltpu.SemaphoreType.DMA,
        ],
    )


_BLK = 1000
_NBLK = _N // _BLK


def _layer_body(h_ref, p0_ref, p1_ref, w1_ref, b1_ref, g1_ref, be1_ref,
                w2_ref, b2_ref, eps_ref, batch_ref, h_out_ref, pooled_ref,
                t_scr, stat_scr, pool_scr):
    ph = pl.program_id(0)
    b = pl.program_id(1)

    @pl.when(ph == 0)
    def _():
        z = ((1.0 + eps_ref[0, 0]) * h_ref[...]
             + p0_ref[...] + p1_ref[...])
        t = jnp.dot(z, w1_ref[...], preferred_element_type=jnp.float32) \
            + b1_ref[...]
        t_scr[pl.ds(b * _BLK, _BLK), :] = t
        part = jnp.concatenate(
            [jnp.sum(t, axis=0, keepdims=True),
             jnp.sum(t * t, axis=0, keepdims=True)], axis=0)

        @pl.when(b == 0)
        def _():
            stat_scr[...] = part

        @pl.when(b > 0)
        def _():
            stat_scr[...] += part

    @pl.when(ph == 1)
    def _():
        mu = stat_scr[0:1, :] * (1.0 / _N)
        var = stat_scr[1:2, :] * (1.0 / _N) - mu * mu
        t = t_scr[pl.ds(b * _BLK, _BLK), :]
        t = (t - mu) * lax.rsqrt(var + 1e-5) * g1_ref[...] + be1_ref[...]
        t = jnp.maximum(t, 0.0)
        u = jnp.dot(t, w2_ref[...], preferred_element_type=jnp.float32) \
            + b2_ref[...]
        u = jnp.maximum(u, 0.0)
        h_out_ref[...] = u
        onehot = (batch_ref[...] ==
                  lax.broadcasted_iota(jnp.int32, (_BLK, _G), 1)
                  ).astype(jnp.float32)
        pp = lax.dot_general(onehot, u, (((0,), (0,)), ((), ())),
                             preferred_element_type=jnp.float32)

        @pl.when(b == 0)
        def _():
            pool_scr[...] = pp

        @pl.when(b > 0)
        def _():
            pool_scr[...] += pp

        @pl.when(b == _NBLK - 1)
        def _():
            pooled_ref[...] = pool_scr[...]


_layer = pl.pallas_call(
    _layer_body,
    grid=(2, _NBLK),
    in_specs=[
        pl.BlockSpec((_BLK, _D), lambda ph, b: ((1 - ph) * b, 0)),
        pl.BlockSpec((_BLK, _D), lambda ph, b: ((1 - ph) * b, 0)),
        pl.BlockSpec((_BLK, _D), lambda ph, b: ((1 - ph) * b, 0)),
        pl.BlockSpec((_D, _D), lambda ph, b: (0, 0)),
        pl.BlockSpec((1, _D), lambda ph, b: (0, 0)),
        pl.BlockSpec((1, _D), lambda ph, b: (0, 0)),
        pl.BlockSpec((1, _D), lambda ph, b: (0, 0)),
        pl.BlockSpec((_D, _D), lambda ph, b: (0, 0)),
        pl.BlockSpec((1, _D), lambda ph, b: (0, 0)),
        pl.BlockSpec((1, 1), lambda ph, b: (0, 0)),
        pl.BlockSpec((_BLK, 1), lambda ph, b: (ph * b, 0)),
    ],
    out_specs=(
        pl.BlockSpec((_BLK, _D), lambda ph, b: (ph * b, 0)),
        pl.BlockSpec((_G, _D), lambda ph, b: (0, 0)),
    ),
    out_shape=(jax.ShapeDtypeStruct((_N, _D), jnp.float32),
               jax.ShapeDtypeStruct((_G, _D), jnp.float32)),
    scratch_shapes=[
        pltpu.VMEM((_N, _D), jnp.float32),
        pltpu.VMEM((2, _D), jnp.float32),
        pltpu.VMEM((_G, _D), jnp.float32),
    ],
)


def _head_body(pc_ref, w1_ref, b1_ref, g_ref, b_ref, w2_ref, b2_ref, out_ref):
    o = jnp.dot(pc_ref[...], w1_ref[...], preferred_element_type=jnp.float32) + b1_ref[...]
    mu = jnp.mean(o, axis=0, keepdims=True)
    var = jnp.mean((o - mu) ** 2, axis=0, keepdims=True)
    o = (o - mu) * lax.rsqrt(var + 1e-5) * g_ref[...] + b_ref[...]
    o = jnp.maximum(o, 0.0)
    o = jnp.dot(o, w2_ref[...], preferred_element_type=jnp.float32) + b2_ref[...]
    m = jnp.max(o, axis=-1, keepdims=True)
    e = jnp.exp(o - m)
    out_ref[...] = o - (m + jnp.log(jnp.sum(e, axis=-1, keepdims=True)))


_head = pl.pallas_call(
    _head_body,
    out_shape=jax.ShapeDtypeStruct((_G, _C), jnp.float32),
)


def kernel(x, edge_index, batch, W1, b1, g1, be1, W2, b2, eps,
           lin1_W, lin1_b, bn_g, bn_b, lin2_W, lin2_b):
    src = edge_index[0]
    dst = edge_index[1].reshape(_NW, _NCHUNK, _K)
    zero = jnp.zeros((_NPAD, _D), jnp.float32)
    batch2d = batch.reshape(_N, 1)
    h = x
    pooled = []
    for i in range(_L):
        p0, p1 = _make_agg()(h, src, dst, zero)
        h, pi = _layer(h, p0, p1, W1[i], b1[i].reshape(1, _D),
                       g1[i].reshape(1, _D), be1[i].reshape(1, _D), W2[i],
                       b2[i].reshape(1, _D), eps[i].reshape(1, 1), batch2d)
        pooled.append(pi)
    pc = jnp.concatenate(pooled, axis=1)
    return _head(pc, lin1_W, lin1_b.reshape(1, _D), bn_g.reshape(1, _D),
                 bn_b.reshape(1, _D), lin2_W, lin2_b.reshape(1, _C))


# speedup vs baseline: 1.0587x; 1.0587x over previous
import functools

import jax
import jax.numpy as jnp
from jax import lax
from jax.experimental import pallas as pl
from jax.experimental.pallas import tpu as pltpu
from jax.experimental.pallas import tpu_sc as plsc

_N = 10000
_E = 320000
_D = 128
_L = 5
_G = 64
_C = 10

_NC = 2
_NS = 16
_NW = _NC * _NS
_EPW = _E // _NW
_K = 80
_NCHUNK = _EPW // _K
_NPAD = 10240
_RPT = _NPAD // _NS


def _agg_body(h_hbm, src_hbm, dst_hbm,
              out0_hbm, out1_hbm,
              acc, src_v, dst_v, rows0, rows1, gsem0, gsem1, zsem):
    c = lax.axis_index("c")
    s = lax.axis_index("s")
    wid = c * _NS + s

    zv = jnp.zeros((16,), jnp.float32)

    def zstore(j, carry):
        rows0[j // 8, pl.ds((j % 8) * 16, 16)] = zv
        return carry

    lax.fori_loop(0, _K * 8, zstore, 0)
    for j in range(_RPT // _K):
        pltpu.async_copy(rows0, acc.at[pl.ds(s * _RPT + j * _K, _K)], zsem)
    cp_src = pltpu.async_copy(src_hbm.at[pl.ds(wid * _EPW, _EPW)], src_v, gsem0)
    cp_dst = pltpu.async_copy(dst_hbm.at[wid], dst_v, gsem1)
    cp_src.wait()
    for j in range(_RPT // _K):
        pltpu.make_async_copy(rows0, acc.at[pl.ds(s * _RPT + j * _K, _K)],
                              zsem).wait()
    pltpu.async_copy(h_hbm.at[src_v.at[pl.ds(0, _K)]], rows0, gsem0)
    cp_dst.wait()
    plsc.subcore_barrier()

    def pair(i, carry):
        t = 2 * i
        pltpu.async_copy(h_hbm.at[src_v.at[pl.ds((t + 1) * _K, _K)]], rows1, gsem1)
        pltpu.make_async_copy(h_hbm.at[src_v.at[pl.ds(t * _K, _K)]], rows0, gsem0).wait()
        pltpu.sync_copy(rows0, acc.at[dst_v.at[t]], add=True)
        pltpu.async_copy(h_hbm.at[src_v.at[pl.ds((t + 2) * _K, _K)]], rows0, gsem0)
        pltpu.make_async_copy(h_hbm.at[src_v.at[pl.ds((t + 1) * _K, _K)]], rows1, gsem1).wait()
        pltpu.sync_copy(rows1, acc.at[dst_v.at[t + 1]], add=True)
        return carry

    lax.fori_loop(0, (_NCHUNK - 1) // 2, pair, 0)
    pltpu.make_async_copy(h_hbm.at[src_v.at[pl.ds((_NCHUNK - 1) * _K, _K)]], rows0, gsem0).wait()
    pltpu.sync_copy(rows0, acc.at[dst_v.at[_NCHUNK - 1]], add=True)
    plsc.subcore_barrier()

    @pl.when(c == 0)
    def _():
        pltpu.sync_copy(acc.at[pl.ds(s * _RPT, _RPT)],
                        out0_hbm.at[pl.ds(s * _RPT, _RPT)])

    @pl.when(c == 1)
    def _():
        pltpu.sync_copy(acc.at[pl.ds(s * _RPT, _RPT)],
                        out1_hbm.at[pl.ds(s * _RPT, _RPT)])


@functools.cache
def _make_agg():
    return pl.kernel(
        _agg_body,
        out_type=(jax.ShapeDtypeStruct((_NPAD, _D), jnp.float32),
                  jax.ShapeDtypeStruct((_NPAD, _D), jnp.float32)),
        mesh=plsc.VectorSubcoreMesh(core_axis_name="c", subcore_axis_name="s",
                                    num_cores=_NC, num_subcores=_NS),
        scratch_types=[
            pltpu.VMEM_SHARED((_NPAD, _D), jnp.float32),
            pltpu.VMEM((_EPW,), jnp.int32),
            pltpu.VMEM((_NCHUNK, _K), jnp.int32),
            pltpu.VMEM((_K, _D), jnp.float32),
            pltpu.VMEM((_K, _D), jnp.float32),
            pltpu.SemaphoreType.DMA,
            pltpu.SemaphoreType.DMA,
            pltpu.SemaphoreType.DMA,
        ],
    )


def _layer_body(h_ref, p0_ref, p1_ref, w1_ref, b1_ref, g1_ref, be1_ref,
                w2_ref, b2_ref, eps_ref, batch_ref, h_out_ref, pooled_ref):
    h = h_ref[...]
    z = (1.0 + eps_ref[0, 0]) * h + p0_ref[...][:_N] + p1_ref[...][:_N]
    t = jnp.dot(z, w1_ref[...], preferred_element_type=jnp.float32) + b1_ref[...]
    mu = jnp.mean(t, axis=0, keepdims=True)
    var = jnp.mean((t - mu) ** 2, axis=0, keepdims=True)
    t = (t - mu) * lax.rsqrt(var + 1e-5) * g1_ref[...] + be1_ref[...]
    t = jnp.maximum(t, 0.0)
    u = jnp.dot(t, w2_ref[...], preferred_element_type=jnp.float32) + b2_ref[...]
    u = jnp.maximum(u, 0.0)
    h_out_ref[...] = u
    onehot = (batch_ref[...] ==
              lax.broadcasted_iota(jnp.int32, (_N, _G), 1)).astype(jnp.float32)
    pooled_ref[...] = lax.dot_general(
        onehot, u, (((0,), (0,)), ((), ())),
        preferred_element_type=jnp.float32)


_layer = pl.pallas_call(
    _layer_body,
    out_shape=(jax.ShapeDtypeStruct((_N, _D), jnp.float32),
               jax.ShapeDtypeStruct((_G, _D), jnp.float32)),
)


def _head_body(pc_ref, w1_ref, b1_ref, g_ref, b_ref, w2_ref, b2_ref, out_ref):
    o = jnp.dot(pc_ref[...], w1_ref[...], preferred_element_type=jnp.float32) + b1_ref[...]
    mu = jnp.mean(o, axis=0, keepdims=True)
    var = jnp.mean((o - mu) ** 2, axis=0, keepdims=True)
    o = (o - mu) * lax.rsqrt(var + 1e-5) * g_ref[...] + b_ref[...]
    o = jnp.maximum(o, 0.0)
    o = jnp.dot(o, w2_ref[...], preferred_element_type=jnp.float32) + b2_ref[...]
    m = jnp.max(o, axis=-1, keepdims=True)
    e = jnp.exp(o - m)
    out_ref[...] = o - (m + jnp.log(jnp.sum(e, axis=-1, keepdims=True)))


_head = pl.pallas_call(
    _head_body,
    out_shape=jax.ShapeDtypeStruct((_G, _C), jnp.float32),
)


def kernel(x, edge_index, batch, W1, b1, g1, be1, W2, b2, eps,
           lin1_W, lin1_b, bn_g, bn_b, lin2_W, lin2_b):
    src = edge_index[0]
    dst = edge_index[1].reshape(_NW, _NCHUNK, _K)
    batch2d = batch.reshape(_N, 1)
    h = x
    pooled = []
    for i in range(_L):
        p0, p1 = _make_agg()(h, src, dst)
        h, pi = _layer(h, p0, p1, W1[i], b1[i].reshape(1, _D),
                       g1[i].reshape(1, _D), be1[i].reshape(1, _D), W2[i],
                       b2[i].reshape(1, _D), eps[i].reshape(1, 1), batch2d)
        pooled.append(pi)
    pc = jnp.concatenate(pooled, axis=1)
    return _head(pc, lin1_W, lin1_b.reshape(1, _D), bn_g.reshape(1, _D),
                 bn_b.reshape(1, _D), lin2_W, lin2_b.reshape(1, _C))


# speedup vs baseline: 1.0692x; 1.0099x over previous
import functools

import jax
import jax.numpy as jnp
from jax import lax
from jax.experimental import pallas as pl
from jax.experimental.pallas import tpu as pltpu
from jax.experimental.pallas import tpu_sc as plsc

_N = 10000
_E = 320000
_D = 128
_L = 5
_G = 64
_C = 10

_NC = 2
_NS = 16
_NW = _NC * _NS
_EPW = _E // _NW
_K = 80
_NCHUNK = _EPW // _K
_NPAD = 10240
_RPT = _NPAD // _NS


def _agg_body(h_hbm, src_hbm, dst_hbm,
              out0_hbm, out1_hbm,
              acc, src_v, dst_v, rows0, rows1, gsem0, gsem1, zsem):
    c = lax.axis_index("c")
    s = lax.axis_index("s")
    wid = c * _NS + s

    zv = jnp.zeros((16,), jnp.float32)

    def zstore(j, carry):
        rows0[j // 8, pl.ds((j % 8) * 16, 16)] = zv
        return carry

    lax.fori_loop(0, _K * 8, zstore, 0)
    for j in range(_RPT // _K):
        pltpu.async_copy(rows0, acc.at[pl.ds(s * _RPT + j * _K, _K)], zsem)
    cp_src = pltpu.async_copy(src_hbm.at[pl.ds(wid * _EPW, _EPW)], src_v, gsem0)
    cp_dst = pltpu.async_copy(dst_hbm.at[wid], dst_v, gsem1)
    cp_src.wait()
    for j in range(_RPT // _K):
        pltpu.make_async_copy(rows0, acc.at[pl.ds(s * _RPT + j * _K, _K)],
                              zsem).wait()
    pltpu.async_copy(h_hbm.at[src_v.at[pl.ds(0, _K)]], rows0, gsem0)
    cp_dst.wait()
    plsc.subcore_barrier()

    def pair(i, carry):
        t = 2 * i
        pltpu.async_copy(h_hbm.at[src_v.at[pl.ds((t + 1) * _K, _K)]], rows1, gsem1)
        pltpu.make_async_copy(h_hbm.at[src_v.at[pl.ds(t * _K, _K)]], rows0, gsem0).wait()
        pltpu.sync_copy(rows0, acc.at[dst_v.at[t]], add=True)
        pltpu.async_copy(h_hbm.at[src_v.at[pl.ds((t + 2) * _K, _K)]], rows0, gsem0)
        pltpu.make_async_copy(h_hbm.at[src_v.at[pl.ds((t + 1) * _K, _K)]], rows1, gsem1).wait()
        pltpu.sync_copy(rows1, acc.at[dst_v.at[t + 1]], add=True)
        return carry

    lax.fori_loop(0, (_NCHUNK - 1) // 2, pair, 0)
    pltpu.make_async_copy(h_hbm.at[src_v.at[pl.ds((_NCHUNK - 1) * _K, _K)]], rows0, gsem0).wait()
    pltpu.sync_copy(rows0, acc.at[dst_v.at[_NCHUNK - 1]], add=True)
    plsc.subcore_barrier()

    @pl.when(c == 0)
    def _():
        pltpu.sync_copy(acc.at[pl.ds(s * _RPT, _RPT)],
                        out0_hbm.at[pl.ds(s * _RPT, _RPT)])

    @pl.when(c == 1)
    def _():
        pltpu.sync_copy(acc.at[pl.ds(s * _RPT, _RPT)],
                        out1_hbm.at[pl.ds(s * _RPT, _RPT)])


@functools.cache
def _make_agg():
    return pl.kernel(
        _agg_body,
        out_type=(jax.ShapeDtypeStruct((_NPAD, _D), jnp.float32),
                  jax.ShapeDtypeStruct((_NPAD, _D), jnp.float32)),
        mesh=plsc.VectorSubcoreMesh(core_axis_name="c", subcore_axis_name="s",
                                    num_cores=_NC, num_subcores=_NS),
        scratch_types=[
            pltpu.VMEM_SHARED((_NPAD, _D), jnp.float32),
            pltpu.VMEM((_EPW,), jnp.int32),
            pltpu.VMEM((_NCHUNK, _K), jnp.int32),
            pltpu.VMEM((_K, _D), jnp.float32),
            pltpu.VMEM((_K, _D), jnp.float32),
            pltpu.SemaphoreType.DMA,
            pltpu.SemaphoreType.DMA,
            pltpu.SemaphoreType.DMA,
        ],
    )


def _layer_body(h_ref, p0_ref, p1_ref, w1_ref, b1_ref, g1_ref, be1_ref,
                w2_ref, b2_ref, eps_ref, batch_ref, h_out_ref, pooled_ref):
    h = h_ref[...]
    z = (1.0 + eps_ref[0, 0]) * h + p0_ref[...][:_N] + p1_ref[...][:_N]
    t = jnp.dot(z, w1_ref[...], preferred_element_type=jnp.float32) + b1_ref[...]
    mu = jnp.mean(t, axis=0, keepdims=True)
    var = jnp.mean((t - mu) ** 2, axis=0, keepdims=True)
    t = (t - mu) * lax.rsqrt(var + 1e-5) * g1_ref[...] + be1_ref[...]
    t = jnp.maximum(t, 0.0)
    u = jnp.dot(t, w2_ref[...], preferred_element_type=jnp.float32) + b2_ref[...]
    u = jnp.maximum(u, 0.0)
    h_out_ref[...] = u
    onehot = (batch_ref[...] ==
              lax.broadcasted_iota(jnp.int32, (_G, _N), 0)).astype(jnp.float32)
    pooled_ref[...] = jnp.dot(onehot, u, preferred_element_type=jnp.float32)


_layer = pl.pallas_call(
    _layer_body,
    out_shape=(jax.ShapeDtypeStruct((_N, _D), jnp.float32),
               jax.ShapeDtypeStruct((_G, _D), jnp.float32)),
)


def _head_body(pc_ref, w1_ref, b1_ref, g_ref, b_ref, w2_ref, b2_ref, out_ref):
    o = jnp.dot(pc_ref[...], w1_ref[...], preferred_element_type=jnp.float32) + b1_ref[...]
    mu = jnp.mean(o, axis=0, keepdims=True)
    var = jnp.mean((o - mu) ** 2, axis=0, keepdims=True)
    o = (o - mu) * lax.rsqrt(var + 1e-5) * g_ref[...] + b_ref[...]
    o = jnp.maximum(o, 0.0)
    o = jnp.dot(o, w2_ref[...], preferred_element_type=jnp.float32) + b2_ref[...]
    m = jnp.max(o, axis=-1, keepdims=True)
    e = jnp.exp(o - m)
    out_ref[...] = o - (m + jnp.log(jnp.sum(e, axis=-1, keepdims=True)))


_head = pl.pallas_call(
    _head_body,
    out_shape=jax.ShapeDtypeStruct((_G, _C), jnp.float32),
)


def kernel(x, edge_index, batch, W1, b1, g1, be1, W2, b2, eps,
           lin1_W, lin1_b, bn_g, bn_b, lin2_W, lin2_b):
    src = edge_index[0]
    dst = edge_index[1].reshape(_NW, _NCHUNK, _K)
    batch2d = batch.reshape(1, _N)
    h = x
    pooled = []
    for i in range(_L):
        p0, p1 = _make_agg()(h, src, dst)
        h, pi = _layer(h, p0, p1, W1[i], b1[i].reshape(1, _D),
                       g1[i].reshape(1, _D), be1[i].reshape(1, _D), W2[i],
                       b2[i].reshape(1, _D), eps[i].reshape(1, 1), batch2d)
        pooled.append(pi)
    pc = jnp.concatenate(pooled, axis=1)
    return _head(pc, lin1_W, lin1_b.reshape(1, _D), bn_g.reshape(1, _D),
                 bn_b.reshape(1, _D), lin2_W, lin2_b.reshape(1, _C))


# speedup vs baseline: 1.0747x; 1.0051x over previous
import functools

import jax
import jax.numpy as jnp
from jax import lax
from jax.experimental import pallas as pl
from jax.experimental.pallas import tpu as pltpu
from jax.experimental.pallas import tpu_sc as plsc

_N = 10000
_E = 320000
_D = 128
_L = 5
_G = 64
_C = 10

_NC = 2
_NS = 16
_NW = _NC * _NS
_EPW = _E // _NW
_K = 80
_NCHUNK = _EPW // _K
_NPAD = 10240
_RPT = _NPAD // _NS


def _agg_body(h_hbm, src_hbm, dst_hbm,
              out0_hbm, out1_hbm,
              acc, src_v, dst_v, rows0, rows1, gsem0, gsem1, zsem, dsem):
    c = lax.axis_index("c")
    s = lax.axis_index("s")
    wid = c * _NS + s

    zv = jnp.zeros((16,), jnp.float32)

    def zstore(j, carry):
        rows0[j // 8, pl.ds((j % 8) * 16, 16)] = zv
        return carry

    lax.fori_loop(0, _K * 8, zstore, 0)
    for j in range(_RPT // _K):
        pltpu.async_copy(rows0, acc.at[pl.ds(s * _RPT + j * _K, _K)], zsem)
    cp_src = pltpu.async_copy(src_hbm.at[pl.ds(wid * _EPW, _EPW)], src_v, gsem0)
    for t in range(4):
        pltpu.async_copy(dst_hbm.at[pl.ds(wid * _EPW + t * _K, _K)],
                         dst_v.at[t], dsem)
    cp_src.wait()
    for j in range(_RPT // _K):
        pltpu.make_async_copy(rows0, acc.at[pl.ds(s * _RPT + j * _K, _K)],
                              zsem).wait()
    pltpu.async_copy(h_hbm.at[src_v.at[pl.ds(0, _K)]], rows0, gsem0)
    plsc.subcore_barrier()

    def _stage_dst(t):
        tc = jnp.minimum(t, _NCHUNK - 1)
        pltpu.async_copy(dst_hbm.at[pl.ds(wid * _EPW + tc * _K, _K)],
                         dst_v.at[tc], dsem)

    def _wait_dst(t):
        pltpu.make_async_copy(dst_hbm.at[pl.ds(wid * _EPW + t * _K, _K)],
                              dst_v.at[t], dsem).wait()

    def pair(i, carry):
        t = 2 * i
        _stage_dst(t + 4)
        _stage_dst(t + 5)
        pltpu.async_copy(h_hbm.at[src_v.at[pl.ds((t + 1) * _K, _K)]], rows1, gsem1)
        pltpu.make_async_copy(h_hbm.at[src_v.at[pl.ds(t * _K, _K)]], rows0, gsem0).wait()
        _wait_dst(t)
        pltpu.sync_copy(rows0, acc.at[dst_v.at[t]], add=True)
        pltpu.async_copy(h_hbm.at[src_v.at[pl.ds((t + 2) * _K, _K)]], rows0, gsem0)
        pltpu.make_async_copy(h_hbm.at[src_v.at[pl.ds((t + 1) * _K, _K)]], rows1, gsem1).wait()
        _wait_dst(t + 1)
        pltpu.sync_copy(rows1, acc.at[dst_v.at[t + 1]], add=True)
        return carry

    lax.fori_loop(0, (_NCHUNK - 1) // 2, pair, 0)
    pltpu.make_async_copy(h_hbm.at[src_v.at[pl.ds((_NCHUNK - 1) * _K, _K)]], rows0, gsem0).wait()
    for _ in range(4):
        _wait_dst(_NCHUNK - 1)
    pltpu.sync_copy(rows0, acc.at[dst_v.at[_NCHUNK - 1]], add=True)
    plsc.subcore_barrier()

    @pl.when(c == 0)
    def _():
        pltpu.sync_copy(acc.at[pl.ds(s * _RPT, _RPT)],
                        out0_hbm.at[pl.ds(s * _RPT, _RPT)])

    @pl.when(c == 1)
    def _():
        pltpu.sync_copy(acc.at[pl.ds(s * _RPT, _RPT)],
                        out1_hbm.at[pl.ds(s * _RPT, _RPT)])


@functools.cache
def _make_agg():
    return pl.kernel(
        _agg_body,
        out_type=(jax.ShapeDtypeStruct((_NPAD, _D), jnp.float32),
                  jax.ShapeDtypeStruct((_NPAD, _D), jnp.float32)),
        mesh=plsc.VectorSubcoreMesh(core_axis_name="c", subcore_axis_name="s",
                                    num_cores=_NC, num_subcores=_NS),
        scratch_types=[
            pltpu.VMEM_SHARED((_NPAD, _D), jnp.float32),
            pltpu.VMEM((_EPW,), jnp.int32),
            pltpu.VMEM((_NCHUNK, _K), jnp.int32),
            pltpu.VMEM((_K, _D), jnp.float32),
            pltpu.VMEM((_K, _D), jnp.float32),
            pltpu.SemaphoreType.DMA,
            pltpu.SemaphoreType.DMA,
            pltpu.SemaphoreType.DMA,
            pltpu.SemaphoreType.DMA,
        ],
    )


def _layer_body(h_ref, p0_ref, p1_ref, w1_ref, b1_ref, g1_ref, be1_ref,
                w2_ref, b2_ref, eps_ref, batch_ref, h_out_ref, pooled_ref):
    h = h_ref[...]
    z = (1.0 + eps_ref[0, 0]) * h + p0_ref[...][:_N] + p1_ref[...][:_N]
    t = jnp.dot(z, w1_ref[...], preferred_element_type=jnp.float32) + b1_ref[...]
    mu = jnp.mean(t, axis=0, keepdims=True)
    var = jnp.mean((t - mu) ** 2, axis=0, keepdims=True)
    t = (t - mu) * lax.rsqrt(var + 1e-5) * g1_ref[...] + be1_ref[...]
    t = jnp.maximum(t, 0.0)
    u = jnp.dot(t, w2_ref[...], preferred_element_type=jnp.float32) + b2_ref[...]
    u = jnp.maximum(u, 0.0)
    h_out_ref[...] = u
    onehot = (batch_ref[...] ==
              lax.broadcasted_iota(jnp.int32, (_G, _N), 0)).astype(jnp.float32)
    pooled_ref[...] = jnp.dot(onehot, u, preferred_element_type=jnp.float32)


_layer = pl.pallas_call(
    _layer_body,
    out_shape=(jax.ShapeDtypeStruct((_N, _D), jnp.float32),
               jax.ShapeDtypeStruct((_G, _D), jnp.float32)),
)


def _head_body(pc_ref, w1_ref, b1_ref, g_ref, b_ref, w2_ref, b2_ref, out_ref):
    o = jnp.dot(pc_ref[...], w1_ref[...], preferred_element_type=jnp.float32) + b1_ref[...]
    mu = jnp.mean(o, axis=0, keepdims=True)
    var = jnp.mean((o - mu) ** 2, axis=0, keepdims=True)
    o = (o - mu) * lax.rsqrt(var + 1e-5) * g_ref[...] + b_ref[...]
    o = jnp.maximum(o, 0.0)
    o = jnp.dot(o, w2_ref[...], preferred_element_type=jnp.float32) + b2_ref[...]
    m = jnp.max(o, axis=-1, keepdims=True)
    e = jnp.exp(o - m)
    out_ref[...] = o - (m + jnp.log(jnp.sum(e, axis=-1, keepdims=True)))


_head = pl.pallas_call(
    _head_body,
    out_shape=jax.ShapeDtypeStruct((_G, _C), jnp.float32),
)


def kernel(x, edge_index, batch, W1, b1, g1, be1, W2, b2, eps,
           lin1_W, lin1_b, bn_g, bn_b, lin2_W, lin2_b):
    src = edge_index[0]
    dst = edge_index[1]
    batch2d = batch.reshape(1, _N)
    h = x
    pooled = []
    for i in range(_L):
        p0, p1 = _make_agg()(h, src, dst)
        h, pi = _layer(h, p0, p1, W1[i], b1[i].reshape(1, _D),
                       g1[i].reshape(1, _D), be1[i].reshape(1, _D), W2[i],
                       b2[i].reshape(1, _D), eps[i].reshape(1, 1), batch2d)
        pooled.append(pi)
    pc = jnp.concatenate(pooled, axis=1)
    return _head(pc, lin1_W, lin1_b.reshape(1, _D), bn_g.reshape(1, _D),
                 bn_b.reshape(1, _D), lin2_W, lin2_b.reshape(1, _C))
